# Initial kernel scaffold; baseline (speedup 1.0000x reference)
import jax
import jax.numpy as jnp
from jax.experimental import pallas as pl


def kernel(x, gate):
    raise NotImplementedError("write your pallas kernel here")


# speedup vs baseline: 6.2204x; 6.2204x over previous
import functools

import jax
import jax.numpy as jnp
from jax.experimental import pallas as pl

MODEL_DIM = 2048
NUM_EXPERTS = 64
TOP_K = 8
N_TOKENS = 16384

BLOCK_TOKENS = 512


def _router_body(x_ref, g_ref, s_ref, t_ref):
    b = x_ref.shape[0]
    e = g_ref.shape[1]
    logits = jnp.dot(x_ref[...], g_ref[...], preferred_element_type=jnp.float32)

    col = jax.lax.broadcasted_iota(jnp.int32, (b, e), 1)
    v = logits
    sel = jnp.zeros((b, e), dtype=jnp.bool_)
    for _ in range(TOP_K):
        m = jnp.max(v, axis=1, keepdims=True)
        is_m = v == m
        first = jnp.min(jnp.where(is_m, col, e), axis=1, keepdims=True)
        pick = col == first
        sel = jnp.logical_or(sel, pick)
        v = jnp.where(pick, -jnp.inf, v)

    out = jnp.where(sel, jax.nn.sigmoid(logits), 0.0)
    s_ref[...] = out.T
    t_ref[...] = pl.program_id(0) * b + jax.lax.broadcasted_iota(
        jnp.int32, (e, b), 1
    )


@functools.partial(jax.jit, static_argnames=())
def kernel(x, gate):
    n, d = x.shape
    e = gate.shape[1]
    grid = (n // BLOCK_TOKENS,)
    scores, token_idx = pl.pallas_call(
        _router_body,
        grid=grid,
        in_specs=[
            pl.BlockSpec((BLOCK_TOKENS, d), lambda i: (i, 0)),
            pl.BlockSpec((d, e), lambda i: (0, 0)),
        ],
        out_specs=[
            pl.BlockSpec((e, BLOCK_TOKENS), lambda i: (0, i)),
            pl.BlockSpec((e, BLOCK_TOKENS), lambda i: (0, i)),
        ],
        out_shape=[
            jax.ShapeDtypeStruct((e, n), jnp.float32),
            jax.ShapeDtypeStruct((e, n), jnp.int32),
        ],
    )(x, gate)
    return (scores, token_idx)


# speedup vs baseline: 10.0785x; 1.6202x over previous
import functools

import jax
import jax.numpy as jnp
from jax.experimental import pallas as pl

MODEL_DIM = 2048
NUM_EXPERTS = 64
TOP_K = 8
N_TOKENS = 16384

BLOCK_TOKENS = 512


def _router_body(x_ref, g_ref, s_ref, t_ref):
    b = x_ref.shape[0]
    e = g_ref.shape[1]
    logits = jnp.dot(x_ref[...], g_ref[...], preferred_element_type=jnp.float32)
    lt = logits.T

    rowf = jax.lax.broadcasted_iota(jnp.int32, (e, b), 0).astype(jnp.float32)
    v = lt
    sel = jnp.zeros((e, b), dtype=jnp.bool_)
    for _ in range(TOP_K):
        m = jnp.max(v, axis=0, keepdims=True)
        is_m = v == m
        first = jnp.min(jnp.where(is_m, rowf, float(e)), axis=0, keepdims=True)
        pick = rowf == first
        sel = jnp.logical_or(sel, pick)
        v = jnp.where(pick, -jnp.inf, v)

    s_ref[...] = jnp.where(sel, jax.nn.sigmoid(lt), 0.0)
    t_ref[...] = pl.program_id(0) * b + jax.lax.broadcasted_iota(
        jnp.int32, (e, b), 1
    )


@functools.partial(jax.jit, static_argnames=())
def kernel(x, gate):
    n, d = x.shape
    e = gate.shape[1]
    grid = (n // BLOCK_TOKENS,)
    scores, token_idx = pl.pallas_call(
        _router_body,
        grid=grid,
        in_specs=[
            pl.BlockSpec((BLOCK_TOKENS, d), lambda i: (i, 0)),
            pl.BlockSpec((d, e), lambda i: (0, 0)),
        ],
        out_specs=[
            pl.BlockSpec((e, BLOCK_TOKENS), lambda i: (0, i)),
            pl.BlockSpec((e, BLOCK_TOKENS), lambda i: (0, i)),
        ],
        out_shape=[
            jax.ShapeDtypeStruct((e, n), jnp.float32),
            jax.ShapeDtypeStruct((e, n), jnp.int32),
        ],
    )(x, gate)
    return (scores, token_idx)


# speedup vs baseline: 12.0088x; 1.1915x over previous
import functools

import jax
import jax.numpy as jnp
from jax.experimental import pallas as pl

MODEL_DIM = 2048
NUM_EXPERTS = 64
TOP_K = 8
N_TOKENS = 16384

BLOCK_TOKENS = 1024


def _router_body(x_ref, g_ref, s_ref, t_ref):
    b = x_ref.shape[0]
    e = g_ref.shape[1]
    logits = jnp.dot(x_ref[...], g_ref[...], preferred_element_type=jnp.float32)
    lt = logits.T

    rowf = jax.lax.broadcasted_iota(jnp.int32, (e, b), 0).astype(jnp.float32)
    v = lt
    sel = jnp.zeros((e, b), dtype=jnp.bool_)
    for _ in range(TOP_K):
        m = jnp.max(v, axis=0, keepdims=True)
        is_m = v == m
        first = jnp.min(jnp.where(is_m, rowf, float(e)), axis=0, keepdims=True)
        pick = rowf == first
        sel = jnp.logical_or(sel, pick)
        v = jnp.where(pick, -jnp.inf, v)

    s_ref[...] = jnp.where(sel, jax.nn.sigmoid(lt), 0.0)
    t_ref[...] = pl.program_id(0) * b + jax.lax.broadcasted_iota(
        jnp.int32, (e, b), 1
    )


@functools.partial(jax.jit, static_argnames=())
def kernel(x, gate):
    n, d = x.shape
    e = gate.shape[1]
    grid = (n // BLOCK_TOKENS,)
    scores, token_idx = pl.pallas_call(
        _router_body,
        grid=grid,
        in_specs=[
            pl.BlockSpec((BLOCK_TOKENS, d), lambda i: (i, 0)),
            pl.BlockSpec((d, e), lambda i: (0, 0)),
        ],
        out_specs=[
            pl.BlockSpec((e, BLOCK_TOKENS), lambda i: (0, i)),
            pl.BlockSpec((e, BLOCK_TOKENS), lambda i: (0, i)),
        ],
        out_shape=[
            jax.ShapeDtypeStruct((e, n), jnp.float32),
            jax.ShapeDtypeStruct((e, n), jnp.int32),
        ],
    )(x, gate)
    return (scores, token_idx)


# speedup vs baseline: 12.4813x; 1.0394x over previous
import functools

import jax
import jax.numpy as jnp
from jax.experimental import pallas as pl

MODEL_DIM = 2048
NUM_EXPERTS = 64
TOP_K = 8
N_TOKENS = 16384

BLOCK_TOKENS = 2048


def _router_body(x_ref, g_ref, s_ref, t_ref):
    b = x_ref.shape[0]
    e = g_ref.shape[1]
    logits = jnp.dot(x_ref[...], g_ref[...], preferred_element_type=jnp.float32)
    lt = logits.T

    rowf = jax.lax.broadcasted_iota(jnp.int32, (e, b), 0).astype(jnp.float32)
    v = lt
    sel = jnp.zeros((e, b), dtype=jnp.bool_)
    for _ in range(TOP_K):
        m = jnp.max(v, axis=0, keepdims=True)
        is_m = v == m
        first = jnp.min(jnp.where(is_m, rowf, float(e)), axis=0, keepdims=True)
        pick = rowf == first
        sel = jnp.logical_or(sel, pick)
        v = jnp.where(pick, -jnp.inf, v)

    s_ref[...] = jnp.where(sel, jax.nn.sigmoid(lt), 0.0)
    t_ref[...] = pl.program_id(0) * b + jax.lax.broadcasted_iota(
        jnp.int32, (e, b), 1
    )


@functools.partial(jax.jit, static_argnames=())
def kernel(x, gate):
    n, d = x.shape
    e = gate.shape[1]
    grid = (n // BLOCK_TOKENS,)
    scores, token_idx = pl.pallas_call(
        _router_body,
        grid=grid,
        in_specs=[
            pl.BlockSpec((BLOCK_TOKENS, d), lambda i: (i, 0)),
            pl.BlockSpec((d, e), lambda i: (0, 0)),
        ],
        out_specs=[
            pl.BlockSpec((e, BLOCK_TOKENS), lambda i: (0, i)),
            pl.BlockSpec((e, BLOCK_TOKENS), lambda i: (0, i)),
        ],
        out_shape=[
            jax.ShapeDtypeStruct((e, n), jnp.float32),
            jax.ShapeDtypeStruct((e, n), jnp.int32),
        ],
    )(x, gate)
    return (scores, token_idx)
